# baseline (speedup 1.0000x reference)
import jax
import jax.numpy as jnp
from jax.experimental import pallas as pl

_BLOCK_S = 256


def _add_kernel(x_ref, emb_ref, out_ref):
    out_ref[...] = x_ref[...] + emb_ref[...][None, :, :]


def kernel(x, emb):
    batch, seq_len, model_dim = x.shape
    bs = _BLOCK_S
    grid = (seq_len // bs,)
    return pl.pallas_call(
        _add_kernel,
        grid=grid,
        in_specs=[
            pl.BlockSpec((batch, bs, model_dim), lambda s: (0, s, 0)),
            pl.BlockSpec((bs, model_dim), lambda s: (s, 0)),
        ],
        out_specs=pl.BlockSpec((batch, bs, model_dim), lambda s: (0, s, 0)),
        out_shape=jax.ShapeDtypeStruct(x.shape, x.dtype),
    )(x, emb)


# speedup vs baseline: 1.0161x; 1.0161x over previous
import jax
import jax.numpy as jnp
from jax.experimental import pallas as pl

_BLOCK_S = 2048


def _add_kernel(x_ref, emb_ref, out_ref):
    out_ref[...] = x_ref[...] + emb_ref[...]


def kernel(x, emb):
    batch, seq_len, model_dim = x.shape
    bs = _BLOCK_S
    n_s = seq_len // bs
    x2 = x.reshape(batch * seq_len, model_dim)
    out2 = pl.pallas_call(
        _add_kernel,
        grid=(n_s, batch),
        in_specs=[
            pl.BlockSpec((bs, model_dim), lambda s, b, n_s=n_s: (b * n_s + s, 0)),
            pl.BlockSpec((bs, model_dim), lambda s, b: (s, 0)),
        ],
        out_specs=pl.BlockSpec((bs, model_dim), lambda s, b, n_s=n_s: (b * n_s + s, 0)),
        out_shape=jax.ShapeDtypeStruct(x2.shape, x.dtype),
    )(x2, emb)
    return out2.reshape(batch, seq_len, model_dim)
